# baseline (speedup 1.0000x reference)
import jax
import jax.numpy as jnp
from jax import lax
from jax.experimental import pallas as pl
from jax.experimental.pallas import tpu as pltpu
from jax.experimental.pallas import tpu_sc as plsc

_N = 10000
_E = 320000
_D = 64
_G = 64
_H = 128
_NX = 128
_NU = 16
_R = 5000
_RP = 5120

_NC = 2
_NS = 16
_NW = _NC * _NS
_EW = _E // _NW
_CH = 128
_EPC = 2560
_NCH = _EPC // _NW
_NP = 10240
_RW = _NP // _NS
_OB = 128

_BN = 2000
_NB = _N // _BN
_BR = 1280
_NBR = _RP // _BR


def _prelu(h, a):
    return jnp.where(h > 0, h, a * h)


def _dg(x, w):
    return lax.dot_general(x.astype(jnp.bfloat16), w.astype(jnp.bfloat16),
                           (((1,), (1,)), ((), ())),
                           preferred_element_type=jnp.float32)


def _enc_body(xr, br, ur, w1r, b1r, a1r, w2r, b2r, a2r, w3r, b3r, outr):
    bb = br[0, 0, :]
    oh = (bb[:, None] == lax.broadcasted_iota(jnp.int32, (_BN, _G), 1)
          ).astype(jnp.bfloat16)
    ub = jnp.dot(oh, ur[...].astype(jnp.bfloat16),
                 preferred_element_type=jnp.float32)
    hcat = jnp.concatenate([xr[...], ub], axis=1)
    h = _prelu(_dg(hcat, w1r[...]) + b1r[...], a1r[0, 0])
    h = _prelu(_dg(h, w2r[...]) + b2r[...], a2r[0, 0])
    outr[...] = _dg(h, w3r[...]) + b3r[...]


def _encode(x, batch3, u, w1, b1, a1, w2, b2, a2, w3, b3):
    def rep(*shape):
        return pl.BlockSpec(shape, lambda i: tuple(0 for _ in shape))
    return pl.pallas_call(
        _enc_body,
        grid=(_NB,),
        in_specs=[
            pl.BlockSpec((_BN, _NX), lambda i: (i, 0)),
            pl.BlockSpec((1, 1, _BN), lambda i: (i, 0, 0)),
            rep(_G, _NU), rep(_H, _NX + _NU), rep(1, _H), rep(1, 1),
            rep(_H, _H), rep(1, _H), rep(1, 1),
            rep(_D, _H), rep(1, _D),
        ],
        out_specs=pl.BlockSpec((_BN, _D), lambda i: (i, 0)),
        out_shape=jax.ShapeDtypeStruct((_N, _D), jnp.float32),
    )(x, batch3, u, w1, b1, a1, w2, b2, a2, w3, b3)


def _lin_body(p0r, p1r, hr, wlr, blr, wrr, outr):
    agg = p0r[0] + p1r[0]
    outr[...] = _dg(agg, wlr[...]) + _dg(hr[...], wrr[...]) + blr[...]


def _sage_linear(parts, h, wl, bl, wr):
    def rep(*shape):
        return pl.BlockSpec(shape, lambda i: tuple(0 for _ in shape))
    blk = pl.BlockSpec((_BN, _D), lambda i: (i, 0))
    return pl.pallas_call(
        _lin_body,
        grid=(_NB,),
        in_specs=[
            pl.BlockSpec((1, _BN, _D), lambda i: (0, i, 0)),
            pl.BlockSpec((1, _BN, _D), lambda i: (1, i, 0)),
            blk, rep(_D, _D), rep(1, _D), rep(_D, _D),
        ],
        out_specs=blk,
        out_shape=jax.ShapeDtypeStruct((_N, _D), jnp.float32),
    )(parts, parts, h, wl, bl, wr)


def _dec_body(p0r, p1r, hr, wlr, blr, wrr,
              w1r, b1r, a1r, w2r, b2r, a2r, w3r, b3r, outr):
    h = (_dg(p0r[...] + p1r[...], wlr[...]) + _dg(hr[...], wrr[...])
         + blr[...])
    h = _prelu(_dg(h, w1r[...]) + b1r[...], a1r[0, 0])
    h = _prelu(_dg(h, w2r[...]) + b2r[...], a2r[0, 0])
    outr[...] = _dg(h, w3r[...]) + b3r[...]


def _decode(p0, p1, h, wl, bl, wr, w1, b1, a1, w2, b2, a2, w3, b3):
    def rep(*shape):
        return pl.BlockSpec(shape, lambda i: tuple(0 for _ in shape))
    blk = pl.BlockSpec((_BR, _D), lambda i: (i, 0))
    return pl.pallas_call(
        _dec_body,
        grid=(_NBR,),
        in_specs=[
            blk, blk, blk, rep(_D, _D), rep(1, _D), rep(_D, _D),
            rep(_H, _D), rep(1, _H), rep(1, 1),
            rep(_H, _H), rep(1, _H), rep(1, 1),
            rep(3, _H), rep(1, 3),
        ],
        out_specs=pl.BlockSpec((_BR, 3), lambda i: (i, 0)),
        out_shape=jax.ShapeDtypeStruct((_RP, 3), jnp.float32),
    )(p0, p1, h, wl, bl, wr, w1, b1, a1, w2, b2, a2, w3, b3)


_NBUF = 5
_GRP = _NCH // _NBUF


def _agg_body(h_hbm, src_hbm, dst_hbm, out_hbm,
              idx_s, idx_d, rows, obuf, agg_sh,
              g0, g1, g2, g3, g4, s0, s1, s2, s3, s4):
    gsems = (g0, g1, g2, g3, g4)
    ssems = (s0, s1, s2, s3, s4)
    c = lax.axis_index("c")
    s = lax.axis_index("s")
    wid = c * _NS + s

    z16 = jnp.zeros((16,), jnp.float32)
    for r in range(_OB):
        for j in range(_D // 16):
            obuf[r, pl.ds(j * 16, 16)] = z16
    for k in range(_RW // _OB):
        pltpu.sync_copy(obuf, agg_sh.at[pl.ds(s * _RW + k * _OB, _OB), :])

    row0 = wid * _NCH
    pltpu.sync_copy(src_hbm.at[pl.ds(row0, _NCH), :], idx_s)
    pltpu.sync_copy(dst_hbm.at[pl.ds(row0, _NCH), :], idx_d)
    plsc.subcore_barrier()

    def wait_gather(b):
        pltpu.make_async_copy(h_hbm.at[idx_s.at[0]], rows.at[b],
                              gsems[b]).wait()

    def wait_scatter(b):
        pltpu.make_async_copy(rows.at[b], agg_sh.at[idx_d.at[0]],
                              ssems[b]).wait()

    for b in range(_NBUF):
        pltpu.async_copy(h_hbm.at[idx_s.at[b]], rows.at[b], gsems[b])

    def group(g, carry):
        base = g * _NBUF
        for b in range(_NBUF):
            wait_gather(b)
            pltpu.async_copy(rows.at[b], agg_sh.at[idx_d.at[base + b]],
                             ssems[b], add=True)
        for b in range(_NBUF):
            wait_scatter(b)
            pltpu.async_copy(h_hbm.at[idx_s.at[base + _NBUF + b]],
                             rows.at[b], gsems[b])
        return carry

    lax.fori_loop(0, _GRP - 1, group, 0)

    base = (_GRP - 1) * _NBUF
    for b in range(_NBUF):
        wait_gather(b)
        pltpu.async_copy(rows.at[b], agg_sh.at[idx_d.at[base + b]],
                         ssems[b], add=True)
    for b in range(_NBUF):
        wait_scatter(b)
    plsc.subcore_barrier()

    for k in range(_RW // _OB):
        pltpu.sync_copy(agg_sh.at[pl.ds(s * _RW + k * _OB, _OB), :], obuf)
        pltpu.sync_copy(
            obuf, out_hbm.at[pl.ds(c * _NP + s * _RW + k * _OB, _OB), :])


def _sage_aggregate(h, src2, dst2):
    f = pl.kernel(
        _agg_body,
        out_type=jax.ShapeDtypeStruct((_NC * _NP, _D), jnp.float32),
        mesh=plsc.VectorSubcoreMesh(core_axis_name="c", subcore_axis_name="s"),
        compiler_params=pltpu.CompilerParams(use_tc_tiling_on_sc=False),
        scratch_types=[
            pltpu.VMEM((_NCH, _CH), jnp.int32),
            pltpu.VMEM((_NCH, _CH), jnp.int32),
            pltpu.VMEM((_NBUF, _CH, _D), jnp.float32),
            pltpu.VMEM((_OB, _D), jnp.float32),
            pltpu.VMEM_SHARED((_NP, _D), jnp.float32),
        ] + [pltpu.SemaphoreType.DMA] * (2 * _NBUF),
    )
    return f(h, src2, dst2)


def _rn_body(parts_hbm, h_hbm, rn_hbm, rnoff_hbm, o0, o1, o2,
             idx, idxo, r0, r1, r2, sem0, sem1, sem2):
    c = lax.axis_index("c")
    s = lax.axis_index("s")
    wid = c * _NS + s
    for k in range(2):
        b = pl.multiple_of(wid * 160 + k * 80, 8)
        pltpu.sync_copy(rn_hbm.at[pl.ds(b, 80)], idx)
        pltpu.sync_copy(rnoff_hbm.at[pl.ds(b, 80)], idxo)
        d0 = pltpu.async_copy(parts_hbm.at[idx], r0, sem0)
        d1 = pltpu.async_copy(parts_hbm.at[idxo], r1, sem1)
        d2 = pltpu.async_copy(h_hbm.at[idx], r2, sem2)
        d0.wait()
        d1.wait()
        d2.wait()
        pltpu.sync_copy(r0, o0.at[pl.ds(b, 80), :])
        pltpu.sync_copy(r1, o1.at[pl.ds(b, 80), :])
        pltpu.sync_copy(r2, o2.at[pl.ds(b, 80), :])


def _rn_gather(parts, h, rn, rnoff):
    sds = jax.ShapeDtypeStruct((_RP, _D), jnp.float32)
    f = pl.kernel(
        _rn_body,
        out_type=(sds, sds, sds),
        mesh=plsc.VectorSubcoreMesh(core_axis_name="c", subcore_axis_name="s"),
        compiler_params=pltpu.CompilerParams(use_tc_tiling_on_sc=False),
        scratch_types=[
            pltpu.VMEM((80,), jnp.int32),
            pltpu.VMEM((80,), jnp.int32),
            pltpu.VMEM((80, _D), jnp.float32),
            pltpu.VMEM((80, _D), jnp.float32),
            pltpu.VMEM((80, _D), jnp.float32),
        ] + [pltpu.SemaphoreType.DMA] * 3,
    )
    return f(parts, h, rn, rnoff)


def kernel(x, u, batch, edge_index, real_nodes,
           enc_W1, enc_b1, enc_a1, enc_W2, enc_b2, enc_a2, enc_W3, enc_b3,
           s1_Wl, s1_bl, s1_Wr, s2_Wl, s2_bl, s2_Wr,
           dec_W1, dec_b1, dec_a1, dec_W2, dec_b2, dec_a2, dec_W3, dec_b3):
    batch3 = batch.astype(jnp.int32).reshape(_NB, 1, _BN)

    h1 = _encode(x, batch3, u, enc_W1,
                 enc_b1.reshape(1, _H), enc_a1.reshape(1, 1),
                 enc_W2, enc_b2.reshape(1, _H), enc_a2.reshape(1, 1),
                 enc_W3, enc_b3.reshape(1, _D))

    npad = _EPC * _CH - _E
    src2 = jnp.concatenate([edge_index[0].astype(jnp.int32),
                            jnp.zeros((npad,), jnp.int32)]).reshape(_EPC, _CH)
    dst2 = jnp.concatenate([edge_index[1].astype(jnp.int32),
                            jnp.full((npad,), _NP - 1, jnp.int32)]
                           ).reshape(_EPC, _CH)

    parts1 = _sage_aggregate(h1, src2, dst2)
    h2 = _sage_linear(parts1.reshape(2, _NP, _D), h1,
                      s1_Wl, s1_bl.reshape(1, _D), s1_Wr)

    parts2 = _sage_aggregate(h2, src2, dst2)

    rn = jnp.concatenate([real_nodes.astype(jnp.int32),
                          jnp.zeros((_RP - _R,), jnp.int32)])
    p0rn, p1rn, h2rn = _rn_gather(parts2, h2, rn, rn + _NP)

    out = _decode(p0rn, p1rn, h2rn,
                  s2_Wl, s2_bl.reshape(1, _D), s2_Wr,
                  dec_W1, dec_b1.reshape(1, _H), dec_a1.reshape(1, 1),
                  dec_W2, dec_b2.reshape(1, _H), dec_a2.reshape(1, 1),
                  dec_W3, dec_b3.reshape(1, 3))
    return out[:_R]


# speedup vs baseline: 2.7328x; 2.7328x over previous
import jax
import jax.numpy as jnp
from jax import lax
from jax.experimental import pallas as pl
from jax.experimental.pallas import tpu as pltpu
from jax.experimental.pallas import tpu_sc as plsc

_N = 10000
_E = 320000
_D = 64
_G = 64
_H = 128
_NX = 128
_NU = 16
_R = 5000
_RP = 5120

_NC = 2
_NS = 16
_NW = _NC * _NS
_EW = _E // _NW
_CH = 80
_NCH = _EW // _CH
_NP = 10240
_RW = _NP // _NS
_OB = 128

_BN = 2000
_NB = _N // _BN
_BR = 1280
_NBR = _RP // _BR


def _prelu(h, a):
    return jnp.where(h > 0, h, a * h)


def _dg(x, w):
    return lax.dot_general(x.astype(jnp.bfloat16), w.astype(jnp.bfloat16),
                           (((1,), (1,)), ((), ())),
                           preferred_element_type=jnp.float32)


def _enc_body(xr, br, ur, w1r, b1r, a1r, w2r, b2r, a2r, w3r, b3r, outr):
    bb = br[0, 0, :]
    oh = (bb[:, None] == lax.broadcasted_iota(jnp.int32, (_BN, _G), 1)
          ).astype(jnp.bfloat16)
    ub = jnp.dot(oh, ur[...].astype(jnp.bfloat16),
                 preferred_element_type=jnp.float32)
    hcat = jnp.concatenate([xr[...], ub], axis=1)
    h = _prelu(_dg(hcat, w1r[...]) + b1r[...], a1r[0, 0])
    h = _prelu(_dg(h, w2r[...]) + b2r[...], a2r[0, 0])
    outr[...] = _dg(h, w3r[...]) + b3r[...]


def _encode(x, batch3, u, w1, b1, a1, w2, b2, a2, w3, b3):
    def rep(*shape):
        return pl.BlockSpec(shape, lambda i: tuple(0 for _ in shape))
    return pl.pallas_call(
        _enc_body,
        grid=(_NB,),
        in_specs=[
            pl.BlockSpec((_BN, _NX), lambda i: (i, 0)),
            pl.BlockSpec((1, 1, _BN), lambda i: (i, 0, 0)),
            rep(_G, _NU), rep(_H, _NX + _NU), rep(1, _H), rep(1, 1),
            rep(_H, _H), rep(1, _H), rep(1, 1),
            rep(_D, _H), rep(1, _D),
        ],
        out_specs=pl.BlockSpec((_BN, _D), lambda i: (i, 0)),
        out_shape=jax.ShapeDtypeStruct((_N, _D), jnp.float32),
    )(x, batch3, u, w1, b1, a1, w2, b2, a2, w3, b3)


def _lin_body(p0r, p1r, hr, wlr, blr, wrr, outr):
    agg = p0r[0] + p1r[0]
    outr[...] = _dg(agg, wlr[...]) + _dg(hr[...], wrr[...]) + blr[...]


def _sage_linear(parts, h, wl, bl, wr):
    def rep(*shape):
        return pl.BlockSpec(shape, lambda i: tuple(0 for _ in shape))
    blk = pl.BlockSpec((_BN, _D), lambda i: (i, 0))
    return pl.pallas_call(
        _lin_body,
        grid=(_NB,),
        in_specs=[
            pl.BlockSpec((1, _BN, _D), lambda i: (0, i, 0)),
            pl.BlockSpec((1, _BN, _D), lambda i: (1, i, 0)),
            blk, rep(_D, _D), rep(1, _D), rep(_D, _D),
        ],
        out_specs=blk,
        out_shape=jax.ShapeDtypeStruct((_N, _D), jnp.float32),
    )(parts, parts, h, wl, bl, wr)


def _dec_body(p0r, p1r, hr, wlr, blr, wrr,
              w1r, b1r, a1r, w2r, b2r, a2r, w3r, b3r, outr):
    h = (_dg(p0r[...] + p1r[...], wlr[...]) + _dg(hr[...], wrr[...])
         + blr[...])
    h = _prelu(_dg(h, w1r[...]) + b1r[...], a1r[0, 0])
    h = _prelu(_dg(h, w2r[...]) + b2r[...], a2r[0, 0])
    outr[...] = _dg(h, w3r[...]) + b3r[...]


def _decode(p0, p1, h, wl, bl, wr, w1, b1, a1, w2, b2, a2, w3, b3):
    def rep(*shape):
        return pl.BlockSpec(shape, lambda i: tuple(0 for _ in shape))
    blk = pl.BlockSpec((_BR, _D), lambda i: (i, 0))
    return pl.pallas_call(
        _dec_body,
        grid=(_NBR,),
        in_specs=[
            blk, blk, blk, rep(_D, _D), rep(1, _D), rep(_D, _D),
            rep(_H, _D), rep(1, _H), rep(1, 1),
            rep(_H, _H), rep(1, _H), rep(1, 1),
            rep(3, _H), rep(1, 3),
        ],
        out_specs=pl.BlockSpec((_BR, 3), lambda i: (i, 0)),
        out_shape=jax.ShapeDtypeStruct((_RP, 3), jnp.float32),
    )(p0, p1, h, wl, bl, wr, w1, b1, a1, w2, b2, a2, w3, b3)


_NBUF = 5
_GRP = _NCH // _NBUF


def _agg_body(h_hbm, e_hbm, out_hbm,
              idx_s, idx_d, rows, obuf, rdbuf, agg_sh,
              g0, g1, g2, g3, g4, s0, s1, s2, s3, s4):
    gsems = (g0, g1, g2, g3, g4)
    ssems = (s0, s1, s2, s3, s4)
    c = lax.axis_index("c")
    s = lax.axis_index("s")
    wid = c * _NS + s

    z16 = jnp.zeros((16,), jnp.float32)
    for r in range(_OB):
        for j in range(_D // 16):
            obuf[r, pl.ds(j * 16, 16)] = z16
    for k in range(_RW // _OB):
        pltpu.sync_copy(obuf, agg_sh.at[pl.ds(s * _RW + k * _OB, _OB), :])

    row0 = wid * _NCH
    pltpu.sync_copy(e_hbm.at[pl.ds(row0, _NCH), :], idx_s)
    pltpu.sync_copy(e_hbm.at[pl.ds(_E // _CH + row0, _NCH), :], idx_d)
    plsc.subcore_barrier()

    def wait_gather(b):
        pltpu.make_async_copy(h_hbm.at[idx_s.at[0]], rows.at[b],
                              gsems[b]).wait()

    def wait_scatter(b):
        pltpu.make_async_copy(rows.at[b], agg_sh.at[idx_d.at[0]],
                              ssems[b]).wait()

    for b in range(_NBUF):
        pltpu.async_copy(h_hbm.at[idx_s.at[b]], rows.at[b], gsems[b])

    def group(g, carry):
        base = g * _NBUF
        for b in range(_NBUF):
            wait_gather(b)
            pltpu.async_copy(rows.at[b], agg_sh.at[idx_d.at[base + b]],
                             ssems[b], add=True)
        for b in range(_NBUF):
            wait_scatter(b)
            pltpu.async_copy(h_hbm.at[idx_s.at[base + _NBUF + b]],
                             rows.at[b], gsems[b])
        return carry

    lax.fori_loop(0, _GRP - 1, group, 0)

    base = (_GRP - 1) * _NBUF
    for b in range(_NBUF):
        wait_gather(b)
        pltpu.async_copy(rows.at[b], agg_sh.at[idx_d.at[base + b]],
                         ssems[b], add=True)
    for b in range(_NBUF):
        wait_scatter(b)
    plsc.subcore_barrier()

    for k in range(2):
        pltpu.sync_copy(agg_sh.at[pl.ds(s * _RW + k * 320, 320), :], rdbuf)
        pltpu.sync_copy(
            rdbuf, out_hbm.at[pl.ds(c * _NP + s * _RW + k * 320, 320), :])


def _sage_aggregate(h, e2):
    f = pl.kernel(
        _agg_body,
        out_type=jax.ShapeDtypeStruct((_NC * _NP, _D), jnp.float32),
        mesh=plsc.VectorSubcoreMesh(core_axis_name="c", subcore_axis_name="s"),
        compiler_params=pltpu.CompilerParams(use_tc_tiling_on_sc=False),
        scratch_types=[
            pltpu.VMEM((_NCH, _CH), jnp.int32),
            pltpu.VMEM((_NCH, _CH), jnp.int32),
            pltpu.VMEM((_NBUF, _CH, _D), jnp.float32),
            pltpu.VMEM((_OB, _D), jnp.float32),
            pltpu.VMEM((320, _D), jnp.float32),
            pltpu.VMEM_SHARED((_NP, _D), jnp.float32),
        ] + [pltpu.SemaphoreType.DMA] * (2 * _NBUF),
    )
    return f(h, e2)


def _rn_body(parts_hbm, h_hbm, rn_hbm, rnoff_hbm, o0, o1, o2,
             idx, idxo, r0, r1, r2, sem0, sem1, sem2):
    c = lax.axis_index("c")
    s = lax.axis_index("s")
    wid = c * _NS + s
    for k in range(2):
        b = pl.multiple_of(wid * 160 + k * 80, 8)
        pltpu.sync_copy(rn_hbm.at[pl.ds(b, 80)], idx)
        pltpu.sync_copy(rnoff_hbm.at[pl.ds(b, 80)], idxo)
        d0 = pltpu.async_copy(parts_hbm.at[idx], r0, sem0)
        d1 = pltpu.async_copy(parts_hbm.at[idxo], r1, sem1)
        d2 = pltpu.async_copy(h_hbm.at[idx], r2, sem2)
        d0.wait()
        d1.wait()
        d2.wait()
        pltpu.sync_copy(r0, o0.at[pl.ds(b, 80), :])
        pltpu.sync_copy(r1, o1.at[pl.ds(b, 80), :])
        pltpu.sync_copy(r2, o2.at[pl.ds(b, 80), :])


def _rn_gather(parts, h, rn, rnoff):
    sds = jax.ShapeDtypeStruct((_RP, _D), jnp.float32)
    f = pl.kernel(
        _rn_body,
        out_type=(sds, sds, sds),
        mesh=plsc.VectorSubcoreMesh(core_axis_name="c", subcore_axis_name="s"),
        compiler_params=pltpu.CompilerParams(use_tc_tiling_on_sc=False),
        scratch_types=[
            pltpu.VMEM((80,), jnp.int32),
            pltpu.VMEM((80,), jnp.int32),
            pltpu.VMEM((80, _D), jnp.float32),
            pltpu.VMEM((80, _D), jnp.float32),
            pltpu.VMEM((80, _D), jnp.float32),
        ] + [pltpu.SemaphoreType.DMA] * 3,
    )
    return f(parts, h, rn, rnoff)


def kernel(x, u, batch, edge_index, real_nodes,
           enc_W1, enc_b1, enc_a1, enc_W2, enc_b2, enc_a2, enc_W3, enc_b3,
           s1_Wl, s1_bl, s1_Wr, s2_Wl, s2_bl, s2_Wr,
           dec_W1, dec_b1, dec_a1, dec_W2, dec_b2, dec_a2, dec_W3, dec_b3):
    batch3 = batch.astype(jnp.int32).reshape(_NB, 1, _BN)

    h1 = _encode(x, batch3, u, enc_W1,
                 enc_b1.reshape(1, _H), enc_a1.reshape(1, 1),
                 enc_W2, enc_b2.reshape(1, _H), enc_a2.reshape(1, 1),
                 enc_W3, enc_b3.reshape(1, _D))

    e2 = edge_index.astype(jnp.int32).reshape(2 * _E // _CH, _CH)

    parts1 = _sage_aggregate(h1, e2)
    h2 = _sage_linear(parts1.reshape(2, _NP, _D), h1,
                      s1_Wl, s1_bl.reshape(1, _D), s1_Wr)

    parts2 = _sage_aggregate(h2, e2)

    rn = jnp.concatenate([real_nodes.astype(jnp.int32),
                          jnp.zeros((_RP - _R,), jnp.int32)])
    p0rn, p1rn, h2rn = _rn_gather(parts2, h2, rn, rn + _NP)

    out = _decode(p0rn, p1rn, h2rn,
                  s2_Wl, s2_bl.reshape(1, _D), s2_Wr,
                  dec_W1, dec_b1.reshape(1, _H), dec_a1.reshape(1, 1),
                  dec_W2, dec_b2.reshape(1, _H), dec_a2.reshape(1, 1),
                  dec_W3, dec_b3.reshape(1, 3))
    return out[:_R]
